# Initial kernel scaffold; baseline (speedup 1.0000x reference)
import jax
import jax.numpy as jnp
from jax.experimental import pallas as pl


def kernel(x, node_structural_feature, edge_attr, edge_weight, W0, b0, msgW1, msgb1, msgW2, msgb2, updW1, updb1, updW2, updb2, edge_index):
    raise NotImplementedError("write your pallas kernel here")


# speedup vs baseline: 3.2670x; 3.2670x over previous
import functools

import jax
import jax.numpy as jnp
from jax import lax
from jax.experimental import pallas as pl
from jax.experimental.pallas import tpu as pltpu
from jax.experimental.pallas import tpu_sc as plsc

N = 10000
E = 320000
D = 128
DS = 128
LAYERS = 3

NC = 2
NS = 16
NW = NC * NS
EW = E // NW
K = 80
NCH = EW // K
NP = 10240
SPT = NP // NS
ZR = 40

_mesh = plsc.VectorSubcoreMesh(core_axis_name="c", subcore_axis_name="s")


@functools.partial(
    pl.kernel,
    mesh=_mesh,
    out_type=jax.ShapeDtypeStruct((NC, NP, DS), jnp.float32),
    scratch_types=[
        pltpu.VMEM((K,), jnp.int32),
        pltpu.VMEM((K,), jnp.int32),
        pltpu.VMEM((K,), jnp.float32),
        pltpu.VMEM((K, D), jnp.float32),
        pltpu.VMEM((K, D), jnp.float32),
        pltpu.VMEM((K, D), jnp.float32),
        pltpu.VMEM((K, DS), jnp.float32),
        pltpu.VMEM((ZR, DS), jnp.float32),
        pltpu.VMEM_SHARED((NP, DS), jnp.float32),
        pltpu.SMEM((K,), jnp.float32),
        pltpu.SemaphoreType.DMA,
        pltpu.SemaphoreType.DMA,
    ],
)
def _sc_edge(a_hbm, b_hbm, c_hbm, src_hbm, dst_hbm, w_hbm, out_hbm,
             srcv, dstv, wv, ga, gb, cc, ov, zb, s_sp, smw, sem_a, sem_b):
    cid = lax.axis_index("c")
    sid = lax.axis_index("s")
    wid = cid * NS + sid

    zv = jnp.zeros((16,), jnp.float32)

    def zrow(r, carry):
        for j in range(DS // 16):
            zb[r, pl.ds(j * 16, 16)] = zv
        return carry

    lax.fori_loop(0, ZR, zrow, 0)
    for t in range(SPT // ZR):
        pltpu.sync_copy(zb, s_sp.at[pl.ds(sid * SPT + t * ZR, ZR)])
    plsc.subcore_barrier()

    def chunk(ch, carry):
        base = wid * EW + ch * K
        pltpu.sync_copy(src_hbm.at[pl.ds(base, K)], srcv)
        pltpu.sync_copy(dst_hbm.at[pl.ds(base, K)], dstv)
        pltpu.sync_copy(w_hbm.at[pl.ds(base, K)], wv)
        pltpu.sync_copy(c_hbm.at[pl.ds(base, K)], cc)
        pltpu.async_copy(a_hbm.at[srcv], ga, sem_a).wait()
        pltpu.async_copy(b_hbm.at[dstv], gb, sem_b).wait()

        def wstage(g, wcarry):
            wvec = wv[pl.ds(g * 16, 16)]
            for l in range(16):
                smw[g * 16 + l] = wvec[l]
            return wcarry

        lax.fori_loop(0, K // 16, wstage, 0)

        def edge(e, ecarry):
            ws = smw[e]
            for j in range(D // 16):
                sl = pl.ds(j * 16, 16)
                v = ga[e, sl] + gb[e, sl] + cc[e, sl]
                ov[e, sl] = jnp.maximum(v, 0.0) * ws
            return ecarry

        lax.fori_loop(0, K, edge, 0)
        pltpu.sync_copy(ov, s_sp.at[dstv], add=True)
        return carry

    lax.fori_loop(0, NCH, chunk, 0)
    plsc.subcore_barrier()

    for t in range(SPT // ZR):
        r0 = sid * SPT + t * ZR
        pltpu.sync_copy(s_sp.at[pl.ds(r0, ZR)], zb)
        pltpu.sync_copy(zb, out_hbm.at[cid, pl.ds(r0, ZR)])


@functools.partial(
    pl.kernel,
    mesh=_mesh,
    out_type=jax.ShapeDtypeStruct((NC, NP, DS), jnp.float32),
    scratch_types=[
        pltpu.VMEM((K,), jnp.int32),
        pltpu.VMEM((K,), jnp.float32),
        pltpu.VMEM((K, DS), jnp.float32),
        pltpu.VMEM((ZR, DS), jnp.float32),
        pltpu.VMEM_SHARED((NP, DS), jnp.float32),
        pltpu.SMEM((K,), jnp.float32),
    ],
)
def _sc_deg(dst_hbm, w_hbm, out_hbm, dstv, wv, ov, zb, s_sp, smw):
    cid = lax.axis_index("c")
    sid = lax.axis_index("s")
    wid = cid * NS + sid

    zv = jnp.zeros((16,), jnp.float32)

    def zrow(r, carry):
        for j in range(DS // 16):
            zb[r, pl.ds(j * 16, 16)] = zv
        return carry

    lax.fori_loop(0, ZR, zrow, 0)
    for t in range(SPT // ZR):
        pltpu.sync_copy(zb, s_sp.at[pl.ds(sid * SPT + t * ZR, ZR)])
    plsc.subcore_barrier()

    def zov(e, carry):
        for j in range(DS // 16):
            ov[e, pl.ds(j * 16, 16)] = zv
        return carry

    lax.fori_loop(0, K, zov, 0)
    lane0 = lax.iota(jnp.int32, 16) == 0

    def chunk(ch, carry):
        base = wid * EW + ch * K
        pltpu.sync_copy(dst_hbm.at[pl.ds(base, K)], dstv)
        pltpu.sync_copy(w_hbm.at[pl.ds(base, K)], wv)

        def wstage(g, wcarry):
            wvec = wv[pl.ds(g * 16, 16)]
            for l in range(16):
                smw[g * 16 + l] = wvec[l]
            return wcarry

        lax.fori_loop(0, K // 16, wstage, 0)

        def edge(e, ecarry):
            ov[e, pl.ds(0, 16)] = jnp.where(lane0, smw[e], 0.0)
            return ecarry

        lax.fori_loop(0, K, edge, 0)
        pltpu.sync_copy(ov, s_sp.at[dstv], add=True)
        return carry

    lax.fori_loop(0, NCH, chunk, 0)
    plsc.subcore_barrier()

    for t in range(SPT // ZR):
        r0 = sid * SPT + t * ZR
        pltpu.sync_copy(s_sp.at[pl.ds(r0, ZR)], zb)
        pltpu.sync_copy(zb, out_hbm.at[cid, pl.ds(r0, ZR)])


BN = 1000
GN = N // BN
BE = 2000
GE = E // BE

_f32 = jnp.float32


def _dot(a, b):
    return jnp.dot(a, b, preferred_element_type=_f32)


def _h0_body(x_ref, w_ref, b_ref, o_ref):
    o_ref[...] = _dot(x_ref[...], w_ref[...]) + b_ref[...]


_tc_h0 = pl.pallas_call(
    _h0_body,
    grid=(GN,),
    in_specs=[pl.BlockSpec((BN, D), lambda i: (i, 0)),
              pl.BlockSpec((D, D), lambda i: (0, 0)),
              pl.BlockSpec((1, D), lambda i: (0, 0))],
    out_specs=pl.BlockSpec((BN, D), lambda i: (i, 0)),
    out_shape=jax.ShapeDtypeStruct((N, D), _f32),
)


def _ab_body(h_ref, sf_ref, ws_ref, wd_ref, wss_ref, wsd_ref, b1_ref,
             a_ref, b_ref):
    h = h_ref[...]
    sf = sf_ref[...]
    a_ref[...] = _dot(h, ws_ref[...]) + _dot(sf, wss_ref[...])
    b_ref[...] = _dot(h, wd_ref[...]) + _dot(sf, wsd_ref[...]) + b1_ref[...]


_tc_ab = pl.pallas_call(
    _ab_body,
    grid=(GN,),
    in_specs=[pl.BlockSpec((BN, D), lambda i: (i, 0)),
              pl.BlockSpec((BN, 8), lambda i: (i, 0)),
              pl.BlockSpec((D, D), lambda i: (0, 0)),
              pl.BlockSpec((D, D), lambda i: (0, 0)),
              pl.BlockSpec((8, D), lambda i: (0, 0)),
              pl.BlockSpec((8, D), lambda i: (0, 0)),
              pl.BlockSpec((1, D), lambda i: (0, 0))],
    out_specs=[pl.BlockSpec((BN, D), lambda i: (i, 0)),
               pl.BlockSpec((BN, D), lambda i: (i, 0))],
    out_shape=[jax.ShapeDtypeStruct((N, D), _f32),
               jax.ShapeDtypeStruct((N, D), _f32)],
)


def _eproj_body(ea_ref, w0_ref, w1_ref, w2_ref, c0_ref, c1_ref, c2_ref):
    ea = ea_ref[...]
    c0_ref[...] = _dot(ea, w0_ref[...])
    c1_ref[...] = _dot(ea, w1_ref[...])
    c2_ref[...] = _dot(ea, w2_ref[...])


_tc_eproj = pl.pallas_call(
    _eproj_body,
    grid=(GE,),
    in_specs=[pl.BlockSpec((BE, 16), lambda i: (i, 0)),
              pl.BlockSpec((16, D), lambda i: (0, 0)),
              pl.BlockSpec((16, D), lambda i: (0, 0)),
              pl.BlockSpec((16, D), lambda i: (0, 0))],
    out_specs=[pl.BlockSpec((BE, D), lambda i: (i, 0)),
               pl.BlockSpec((BE, D), lambda i: (i, 0)),
               pl.BlockSpec((BE, D), lambda i: (i, 0))],
    out_shape=[jax.ShapeDtypeStruct((E, D), _f32),
               jax.ShapeDtypeStruct((E, D), _f32),
               jax.ShapeDtypeStruct((E, D), _f32)],
)


def _upd_body(h_ref, sp_ref, dg_ref, w2_ref, b2_ref, u1h_ref, u1u_ref,
              ub1_ref, u2_ref, ub2_ref, o_ref):
    s = sp_ref[0] + sp_ref[1]
    deg = dg_ref[0, :, 0:1] + dg_ref[1, :, 0:1]
    upd = _dot(s, w2_ref[...]) + deg * b2_ref[...]
    t = _dot(h_ref[...], u1h_ref[...]) + _dot(upd, u1u_ref[...]) + ub1_ref[...]
    t = jnp.maximum(t, 0.0)
    o = _dot(t, u2_ref[...]) + ub2_ref[...]
    o_ref[...] = jnp.maximum(o, 0.0)


_tc_update = pl.pallas_call(
    _upd_body,
    grid=(GN,),
    in_specs=[pl.BlockSpec((BN, D), lambda i: (i, 0)),
              pl.BlockSpec((NC, BN, DS), lambda i: (0, i, 0)),
              pl.BlockSpec((NC, BN, DS), lambda i: (0, i, 0)),
              pl.BlockSpec((D, D), lambda i: (0, 0)),
              pl.BlockSpec((1, D), lambda i: (0, 0)),
              pl.BlockSpec((D, D), lambda i: (0, 0)),
              pl.BlockSpec((D, D), lambda i: (0, 0)),
              pl.BlockSpec((1, D), lambda i: (0, 0)),
              pl.BlockSpec((D, D), lambda i: (0, 0)),
              pl.BlockSpec((1, D), lambda i: (0, 0))],
    out_specs=pl.BlockSpec((BN, D), lambda i: (i, 0)),
    out_shape=jax.ShapeDtypeStruct((N, D), _f32),
)


def _rs_body(h_ref, o_ref):
    @pl.when(pl.program_id(0) == 0)
    def _():
        o_ref[...] = jnp.zeros_like(o_ref)

    o_ref[...] += jnp.sum(h_ref[...], axis=0, keepdims=True)


_tc_rowsum = pl.pallas_call(
    _rs_body,
    grid=(GN,),
    in_specs=[pl.BlockSpec((BN, D), lambda i: (i, 0))],
    out_specs=pl.BlockSpec((1, D), lambda i: (0, 0)),
    out_shape=jax.ShapeDtypeStruct((1, D), _f32),
)


def kernel(x, node_structural_feature, edge_attr, edge_weight, W0, b0,
           msgW1, msgb1, msgW2, msgb2, updW1, updb1, updW2, updb2,
           edge_index):
    src = edge_index[0]
    dst = edge_index[1]
    sfp = jnp.pad(node_structural_feature, ((0, 0), (0, 2)))

    h = _tc_h0(x, W0, b0.reshape(1, D))
    c_all = _tc_eproj(edge_attr,
                      msgW1[0, 268:284], msgW1[1, 268:284], msgW1[2, 268:284])
    degp = _sc_deg(dst, edge_weight)

    for i in range(LAYERS):
        ws = msgW1[i, 0:128]
        wd = msgW1[i, 128:256]
        wss = jnp.pad(msgW1[i, 256:262], ((0, 2), (0, 0)))
        wsd = jnp.pad(msgW1[i, 262:268], ((0, 2), (0, 0)))
        a, b = _tc_ab(h, sfp, ws, wd, wss, wsd, msgb1[i].reshape(1, D))
        sp = _sc_edge(a, b, c_all[i], src, dst, edge_weight)
        h = _tc_update(h, sp, degp, msgW2[i], msgb2[i].reshape(1, D),
                       updW1[i, 0:128], updW1[i, 128:256],
                       updb1[i].reshape(1, D), updW2[i],
                       updb2[i].reshape(1, D))

    graph_feature = _tc_rowsum(h)
    return graph_feature, h


# speedup vs baseline: 4.9265x; 1.5080x over previous
import functools

import jax
import jax.numpy as jnp
from jax import lax
from jax.experimental import pallas as pl
from jax.experimental.pallas import tpu as pltpu
from jax.experimental.pallas import tpu_sc as plsc

N = 10000
E = 320000
D = 128
DS = 128
LAYERS = 3

NC = 2
NS = 16
NW = NC * NS
EW = E // NW
K = 40
NCH = EW // K
KD = 80
NCHD = EW // KD
NP = 10240
SPT = NP // NS
ZR = 40

_mesh = plsc.VectorSubcoreMesh(core_axis_name="c", subcore_axis_name="s")


@functools.partial(
    pl.kernel,
    mesh=_mesh,
    out_type=jax.ShapeDtypeStruct((NC, NP, DS), jnp.float32),
    scratch_types=(
        [pltpu.VMEM((K,), jnp.int32)] * 4 +
        [pltpu.VMEM((K,), jnp.float32)] * 2 +
        [pltpu.VMEM((K, D), jnp.float32)] * 8 +
        [pltpu.VMEM_SHARED((NP, DS), jnp.float32),
         pltpu.SMEM((K,), jnp.float32)] +
        [pltpu.SemaphoreType.DMA] * 4
    ),
)
def _sc_edge(a_hbm, b_hbm, c_hbm, src_hbm, dst_hbm, w_hbm, out_hbm,
             srcv0, srcv1, dstv0, dstv1, wv0, wv1,
             ga0, ga1, gb0, gb1, cc0, cc1, ov0, ov1,
             s_sp, smw, semL0, semL1, semG0, semG1):
    srcv = (srcv0, srcv1)
    dstv = (dstv0, dstv1)
    wv = (wv0, wv1)
    ga = (ga0, ga1)
    gb = (gb0, gb1)
    cc = (cc0, cc1)
    ov = (ov0, ov1)
    semL = (semL0, semL1)
    semG = (semG0, semG1)
    cid = lax.axis_index("c")
    sid = lax.axis_index("s")
    wid = cid * NS + sid
    ebase = wid * EW

    zv = jnp.zeros((16,), jnp.float32)

    def zrow(r, carry):
        for j in range(DS // 16):
            ov0[r, pl.ds(j * 16, 16)] = zv
        return carry

    lax.fori_loop(0, K, zrow, 0)
    for t in range(SPT // K):
        pltpu.sync_copy(ov0, s_sp.at[pl.ds(sid * SPT + t * K, K)])
    plsc.subcore_barrier()

    def lin_issue(ch, s):
        base = ebase + ch * K
        pltpu.async_copy(src_hbm.at[pl.ds(base, K)], srcv[s], semL[s])
        pltpu.async_copy(dst_hbm.at[pl.ds(base, K)], dstv[s], semL[s])
        pltpu.async_copy(w_hbm.at[pl.ds(base, K)], wv[s], semL[s])
        pltpu.async_copy(c_hbm.at[pl.ds(base, K)], cc[s], semL[s])

    def lin_wait(s):
        pltpu.make_async_copy(src_hbm.at[pl.ds(0, K)], srcv[s], semL[s]).wait()
        pltpu.make_async_copy(dst_hbm.at[pl.ds(0, K)], dstv[s], semL[s]).wait()
        pltpu.make_async_copy(w_hbm.at[pl.ds(0, K)], wv[s], semL[s]).wait()
        pltpu.make_async_copy(c_hbm.at[pl.ds(0, K)], cc[s], semL[s]).wait()

    def gath_issue(s):
        pltpu.async_copy(a_hbm.at[srcv[s]], ga[s], semG[s])
        pltpu.async_copy(b_hbm.at[dstv[s]], gb[s], semG[s])

    def gath_wait(s):
        pltpu.make_async_copy(a_hbm.at[srcv[s]], ga[s], semG[s]).wait()
        pltpu.make_async_copy(b_hbm.at[dstv[s]], gb[s], semG[s]).wait()

    def compute_scatter(s):
        def wstage(g, wcarry):
            wvec = wv[s][pl.ds(g * 16, 16)]
            for l in range(16):
                smw[g * 16 + l] = wvec[l]
            return wcarry

        lax.fori_loop(0, K // 16, wstage, 0)

        def edge(e, ecarry):
            ws = smw[e]
            for j in range(D // 16):
                sl = pl.ds(j * 16, 16)
                v = ga[s][e, sl] + gb[s][e, sl] + cc[s][e, sl]
                ov[s][e, sl] = jnp.maximum(v, 0.0) * ws
            return ecarry

        lax.fori_loop(0, K, edge, 0)
        pltpu.sync_copy(ov[s], s_sp.at[dstv[s]], add=True)

    lin_issue(0, 0)
    lin_wait(0)
    gath_issue(0)
    lin_issue(1, 1)

    def piped(ch, s):
        gath_wait(s)
        lin_wait(1 - s)
        gath_issue(1 - s)
        compute_scatter(s)
        lin_issue(ch + 2, s)

    def outer(g, carry):
        piped(g * 2, 0)
        piped(g * 2 + 1, 1)
        return carry

    lax.fori_loop(0, (NCH - 2) // 2, outer, 0)
    gath_wait(0)
    lin_wait(1)
    gath_issue(1)
    compute_scatter(0)
    gath_wait(1)
    compute_scatter(1)

    plsc.subcore_barrier()

    for t in range(SPT // K):
        r0 = sid * SPT + t * K
        pltpu.sync_copy(s_sp.at[pl.ds(r0, K)], ga0)
        pltpu.sync_copy(ga0, out_hbm.at[cid, pl.ds(r0, K)])


@functools.partial(
    pl.kernel,
    mesh=_mesh,
    out_type=jax.ShapeDtypeStruct((NC, NP, DS), jnp.float32),
    scratch_types=[
        pltpu.VMEM((KD,), jnp.int32),
        pltpu.VMEM((KD,), jnp.float32),
        pltpu.VMEM((KD, DS), jnp.float32),
        pltpu.VMEM((ZR, DS), jnp.float32),
        pltpu.VMEM_SHARED((NP, DS), jnp.float32),
        pltpu.SMEM((KD,), jnp.float32),
    ],
)
def _sc_deg(dst_hbm, w_hbm, out_hbm, dstv, wv, ov, zb, s_sp, smw):
    cid = lax.axis_index("c")
    sid = lax.axis_index("s")
    wid = cid * NS + sid

    zv = jnp.zeros((16,), jnp.float32)

    def zrow(r, carry):
        for j in range(DS // 16):
            zb[r, pl.ds(j * 16, 16)] = zv
        return carry

    lax.fori_loop(0, ZR, zrow, 0)
    for t in range(SPT // ZR):
        pltpu.sync_copy(zb, s_sp.at[pl.ds(sid * SPT + t * ZR, ZR)])
    plsc.subcore_barrier()

    def zov(e, carry):
        for j in range(DS // 16):
            ov[e, pl.ds(j * 16, 16)] = zv
        return carry

    lax.fori_loop(0, KD, zov, 0)
    lane0 = lax.iota(jnp.int32, 16) == 0

    def chunk(ch, carry):
        base = wid * EW + ch * KD
        pltpu.sync_copy(dst_hbm.at[pl.ds(base, KD)], dstv)
        pltpu.sync_copy(w_hbm.at[pl.ds(base, KD)], wv)

        def wstage(g, wcarry):
            wvec = wv[pl.ds(g * 16, 16)]
            for l in range(16):
                smw[g * 16 + l] = wvec[l]
            return wcarry

        lax.fori_loop(0, KD // 16, wstage, 0)

        def edge(e, ecarry):
            ov[e, pl.ds(0, 16)] = jnp.where(lane0, smw[e], 0.0)
            return ecarry

        lax.fori_loop(0, KD, edge, 0)
        pltpu.sync_copy(ov, s_sp.at[dstv], add=True)
        return carry

    lax.fori_loop(0, NCHD, chunk, 0)
    plsc.subcore_barrier()

    for t in range(SPT // ZR):
        r0 = sid * SPT + t * ZR
        pltpu.sync_copy(s_sp.at[pl.ds(r0, ZR)], zb)
        pltpu.sync_copy(zb, out_hbm.at[cid, pl.ds(r0, ZR)])


BN = 1000
GN = N // BN
BE = 2000
GE = E // BE

_f32 = jnp.float32


def _dot(a, b):
    return jnp.dot(a, b, preferred_element_type=_f32)


def _h0_body(x_ref, w_ref, b_ref, o_ref):
    o_ref[...] = _dot(x_ref[...], w_ref[...]) + b_ref[...]


_tc_h0 = pl.pallas_call(
    _h0_body,
    grid=(GN,),
    in_specs=[pl.BlockSpec((BN, D), lambda i: (i, 0)),
              pl.BlockSpec((D, D), lambda i: (0, 0)),
              pl.BlockSpec((1, D), lambda i: (0, 0))],
    out_specs=pl.BlockSpec((BN, D), lambda i: (i, 0)),
    out_shape=jax.ShapeDtypeStruct((N, D), _f32),
)


def _ab_body(h_ref, sf_ref, ws_ref, wd_ref, wss_ref, wsd_ref, b1_ref,
             a_ref, b_ref):
    h = h_ref[...]
    sf = sf_ref[...]
    a_ref[...] = _dot(h, ws_ref[...]) + _dot(sf, wss_ref[...])
    b_ref[...] = _dot(h, wd_ref[...]) + _dot(sf, wsd_ref[...]) + b1_ref[...]


_tc_ab = pl.pallas_call(
    _ab_body,
    grid=(GN,),
    in_specs=[pl.BlockSpec((BN, D), lambda i: (i, 0)),
              pl.BlockSpec((BN, 8), lambda i: (i, 0)),
              pl.BlockSpec((D, D), lambda i: (0, 0)),
              pl.BlockSpec((D, D), lambda i: (0, 0)),
              pl.BlockSpec((8, D), lambda i: (0, 0)),
              pl.BlockSpec((8, D), lambda i: (0, 0)),
              pl.BlockSpec((1, D), lambda i: (0, 0))],
    out_specs=[pl.BlockSpec((BN, D), lambda i: (i, 0)),
               pl.BlockSpec((BN, D), lambda i: (i, 0))],
    out_shape=[jax.ShapeDtypeStruct((N, D), _f32),
               jax.ShapeDtypeStruct((N, D), _f32)],
)


def _eproj_body(ea_ref, w0_ref, w1_ref, w2_ref, c0_ref, c1_ref, c2_ref):
    ea = ea_ref[...]
    c0_ref[...] = _dot(ea, w0_ref[...])
    c1_ref[...] = _dot(ea, w1_ref[...])
    c2_ref[...] = _dot(ea, w2_ref[...])


_tc_eproj = pl.pallas_call(
    _eproj_body,
    grid=(GE,),
    in_specs=[pl.BlockSpec((BE, 16), lambda i: (i, 0)),
              pl.BlockSpec((16, D), lambda i: (0, 0)),
              pl.BlockSpec((16, D), lambda i: (0, 0)),
              pl.BlockSpec((16, D), lambda i: (0, 0))],
    out_specs=[pl.BlockSpec((BE, D), lambda i: (i, 0)),
               pl.BlockSpec((BE, D), lambda i: (i, 0)),
               pl.BlockSpec((BE, D), lambda i: (i, 0))],
    out_shape=[jax.ShapeDtypeStruct((E, D), _f32),
               jax.ShapeDtypeStruct((E, D), _f32),
               jax.ShapeDtypeStruct((E, D), _f32)],
)


def _upd_body(h_ref, sp_ref, dg_ref, w2_ref, b2_ref, u1h_ref, u1u_ref,
              ub1_ref, u2_ref, ub2_ref, o_ref):
    s = sp_ref[0] + sp_ref[1]
    deg = dg_ref[0, :, 0:1] + dg_ref[1, :, 0:1]
    upd = _dot(s, w2_ref[...]) + deg * b2_ref[...]
    t = _dot(h_ref[...], u1h_ref[...]) + _dot(upd, u1u_ref[...]) + ub1_ref[...]
    t = jnp.maximum(t, 0.0)
    o = _dot(t, u2_ref[...]) + ub2_ref[...]
    o_ref[...] = jnp.maximum(o, 0.0)


_tc_update = pl.pallas_call(
    _upd_body,
    grid=(GN,),
    in_specs=[pl.BlockSpec((BN, D), lambda i: (i, 0)),
              pl.BlockSpec((NC, BN, DS), lambda i: (0, i, 0)),
              pl.BlockSpec((NC, BN, DS), lambda i: (0, i, 0)),
              pl.BlockSpec((D, D), lambda i: (0, 0)),
              pl.BlockSpec((1, D), lambda i: (0, 0)),
              pl.BlockSpec((D, D), lambda i: (0, 0)),
              pl.BlockSpec((D, D), lambda i: (0, 0)),
              pl.BlockSpec((1, D), lambda i: (0, 0)),
              pl.BlockSpec((D, D), lambda i: (0, 0)),
              pl.BlockSpec((1, D), lambda i: (0, 0))],
    out_specs=pl.BlockSpec((BN, D), lambda i: (i, 0)),
    out_shape=jax.ShapeDtypeStruct((N, D), _f32),
)


def _rs_body(h_ref, o_ref):
    @pl.when(pl.program_id(0) == 0)
    def _():
        o_ref[...] = jnp.zeros_like(o_ref)

    o_ref[...] += jnp.sum(h_ref[...], axis=0, keepdims=True)


_tc_rowsum = pl.pallas_call(
    _rs_body,
    grid=(GN,),
    in_specs=[pl.BlockSpec((BN, D), lambda i: (i, 0))],
    out_specs=pl.BlockSpec((1, D), lambda i: (0, 0)),
    out_shape=jax.ShapeDtypeStruct((1, D), _f32),
)


def kernel(x, node_structural_feature, edge_attr, edge_weight, W0, b0,
           msgW1, msgb1, msgW2, msgb2, updW1, updb1, updW2, updb2,
           edge_index):
    src = edge_index[0]
    dst = edge_index[1]
    sfp = jnp.pad(node_structural_feature, ((0, 0), (0, 2)))

    h = _tc_h0(x, W0, b0.reshape(1, D))
    c_all = _tc_eproj(edge_attr,
                      msgW1[0, 268:284], msgW1[1, 268:284], msgW1[2, 268:284])
    degp = _sc_deg(dst, edge_weight)

    for i in range(LAYERS):
        ws = msgW1[i, 0:128]
        wd = msgW1[i, 128:256]
        wss = jnp.pad(msgW1[i, 256:262], ((0, 2), (0, 0)))
        wsd = jnp.pad(msgW1[i, 262:268], ((0, 2), (0, 0)))
        a, b = _tc_ab(h, sfp, ws, wd, wss, wsd, msgb1[i].reshape(1, D))
        sp = _sc_edge(a, b, c_all[i], src, dst, edge_weight)
        h = _tc_update(h, sp, degp, msgW2[i], msgb2[i].reshape(1, D),
                       updW1[i, 0:128], updW1[i, 128:256],
                       updb1[i].reshape(1, D), updW2[i],
                       updb2[i].reshape(1, D))

    graph_feature = _tc_rowsum(h)
    return graph_feature, h


# speedup vs baseline: 5.7574x; 1.1686x over previous
import functools

import jax
import jax.numpy as jnp
from jax import lax
from jax.experimental import pallas as pl
from jax.experimental.pallas import tpu as pltpu
from jax.experimental.pallas import tpu_sc as plsc

N = 10000
E = 320000
D = 128
DS = 128
LAYERS = 3

NC = 2
NS = 16
NW = NC * NS
EW = E // NW
K = 40
NCH = EW // K
KD = 40
NCHD = EW // KD
NP = 10240
SPT = NP // NS
ZR = 40

_mesh = plsc.VectorSubcoreMesh(core_axis_name="c", subcore_axis_name="s")


@functools.partial(
    pl.kernel,
    mesh=_mesh,
    out_type=jax.ShapeDtypeStruct((NC, NP, DS), jnp.float32),
    scratch_types=(
        [pltpu.VMEM((K,), jnp.int32)] * 6 +
        [pltpu.VMEM((K,), jnp.float32)] * 2 +
        [pltpu.VMEM((K, D), jnp.float32)] * 8 +
        [pltpu.VMEM_SHARED((NP, DS), jnp.float32),
         pltpu.SMEM((K,), jnp.float32)] +
        [pltpu.SemaphoreType.DMA] * 6
    ),
)
def _sc_edge(a_hbm, b_hbm, c_hbm, src_hbm, dst_hbm, w_hbm, out_hbm,
             srcv0, srcv1, dstv0, dstv1, dsts0, dsts1, wv0, wv1,
             ga0, ga1, gb0, gb1, cc0, cc1, ov0, ov1,
             s_sp, smw, semL0, semL1, semG0, semG1, semS0, semS1):
    srcv = (srcv0, srcv1)
    dstv = (dstv0, dstv1)
    dsts = (dsts0, dsts1)
    wv = (wv0, wv1)
    ga = (ga0, ga1)
    gb = (gb0, gb1)
    cc = (cc0, cc1)
    ov = (ov0, ov1)
    semL = (semL0, semL1)
    semG = (semG0, semG1)
    semS = (semS0, semS1)
    cid = lax.axis_index("c")
    sid = lax.axis_index("s")
    wid = cid * NS + sid
    ebase = wid * EW

    zv = jnp.zeros((16,), jnp.float32)

    def zrow(r, carry):
        for j in range(DS // 16):
            ov0[r, pl.ds(j * 16, 16)] = zv
        return carry

    lax.fori_loop(0, K, zrow, 0)
    for t in range(SPT // K):
        pltpu.sync_copy(ov0, s_sp.at[pl.ds(sid * SPT + t * K, K)])
    plsc.subcore_barrier()

    def lin_issue(ch, s):
        base = ebase + ch * K
        pltpu.async_copy(src_hbm.at[pl.ds(base, K)], srcv[s], semL[s])
        pltpu.async_copy(dst_hbm.at[pl.ds(base, K)], dstv[s], semL[s])
        pltpu.async_copy(w_hbm.at[pl.ds(base, K)], wv[s], semL[s])
        pltpu.async_copy(c_hbm.at[pl.ds(base, K)], cc[s], semL[s])

    def lin_wait(s):
        pltpu.make_async_copy(src_hbm.at[pl.ds(0, K)], srcv[s], semL[s]).wait()
        pltpu.make_async_copy(dst_hbm.at[pl.ds(0, K)], dstv[s], semL[s]).wait()
        pltpu.make_async_copy(w_hbm.at[pl.ds(0, K)], wv[s], semL[s]).wait()
        pltpu.make_async_copy(c_hbm.at[pl.ds(0, K)], cc[s], semL[s]).wait()

    def gath_issue(s):
        pltpu.async_copy(a_hbm.at[srcv[s]], ga[s], semG[s])
        pltpu.async_copy(b_hbm.at[dstv[s]], gb[s], semG[s])

    def gath_wait(s):
        pltpu.make_async_copy(a_hbm.at[srcv[s]], ga[s], semG[s]).wait()
        pltpu.make_async_copy(b_hbm.at[dstv[s]], gb[s], semG[s]).wait()

    def compute(s):
        for base in (0, 16, K - 16):
            wvec = wv[s][pl.ds(base, 16)]
            for l in range(16):
                smw[base + l] = wvec[l]
        dsts[s][pl.ds(0, 16)] = dstv[s][pl.ds(0, 16)]
        dsts[s][pl.ds(16, 16)] = dstv[s][pl.ds(16, 16)]
        dsts[s][pl.ds(K - 16, 16)] = dstv[s][pl.ds(K - 16, 16)]

        def edge(e, ecarry):
            ws = smw[e]
            for j in range(D // 16):
                sl = pl.ds(j * 16, 16)
                v = ga[s][e, sl] + gb[s][e, sl] + cc[s][e, sl]
                ov[s][e, sl] = jnp.maximum(v, 0.0) * ws
            return ecarry

        lax.fori_loop(0, K, edge, 0)

    def scat_issue(s):
        pltpu.async_copy(ov[s], s_sp.at[dsts[s]], semS[s], add=True)

    def scat_wait(s):
        pltpu.make_async_copy(ov[s], s_sp.at[dsts[s]], semS[s]).wait()

    lin_issue(0, 0)
    lin_wait(0)
    gath_issue(0)
    lin_issue(1, 1)

    def piped(ch, s, wait_scat):
        gath_wait(s)
        lin_wait(1 - s)
        gath_issue(1 - s)
        if wait_scat:
            scat_wait(s)
        compute(s)
        scat_issue(s)
        lin_issue(ch + 2, s)

    piped(0, 0, False)
    piped(1, 1, False)

    def outer(g, carry):
        piped(2 + g * 2, 0, True)
        piped(3 + g * 2, 1, True)
        return carry

    lax.fori_loop(0, (NCH - 4) // 2, outer, 0)
    gath_wait(0)
    lin_wait(1)
    gath_issue(1)
    scat_wait(0)
    compute(0)
    scat_issue(0)
    gath_wait(1)
    scat_wait(1)
    compute(1)
    scat_issue(1)
    scat_wait(0)
    scat_wait(1)

    plsc.subcore_barrier()

    for t in range(SPT // K):
        r0 = sid * SPT + t * K
        pltpu.sync_copy(s_sp.at[pl.ds(r0, K)], ga0)
        pltpu.sync_copy(ga0, out_hbm.at[cid, pl.ds(r0, K)])


@functools.partial(
    pl.kernel,
    mesh=_mesh,
    out_type=jax.ShapeDtypeStruct((NC, NP, DS), jnp.float32),
    scratch_types=(
        [pltpu.VMEM((KD,), jnp.int32)] * 4 +
        [pltpu.VMEM((KD,), jnp.float32)] * 2 +
        [pltpu.VMEM((KD, DS), jnp.float32)] * 2 +
        [pltpu.VMEM((ZR, DS), jnp.float32),
         pltpu.VMEM_SHARED((NP, DS), jnp.float32),
         pltpu.SMEM((KD,), jnp.float32)] +
        [pltpu.SemaphoreType.DMA] * 4
    ),
)
def _sc_deg(dst_hbm, w_hbm, out_hbm, dstv0, dstv1, dsts0, dsts1, wv0, wv1,
            ov0, ov1, zb, s_sp, smw, semL0, semL1, semS0, semS1):
    dstv = (dstv0, dstv1)
    dsts = (dsts0, dsts1)
    wv = (wv0, wv1)
    ov = (ov0, ov1)
    semL = (semL0, semL1)
    semS = (semS0, semS1)
    cid = lax.axis_index("c")
    sid = lax.axis_index("s")
    wid = cid * NS + sid
    ebase = wid * EW

    zv = jnp.zeros((16,), jnp.float32)

    def zrow(r, carry):
        for j in range(DS // 16):
            zb[r, pl.ds(j * 16, 16)] = zv
        return carry

    lax.fori_loop(0, ZR, zrow, 0)
    for t in range(SPT // ZR):
        pltpu.sync_copy(zb, s_sp.at[pl.ds(sid * SPT + t * ZR, ZR)])
    plsc.subcore_barrier()

    def zov(e, carry):
        for j in range(DS // 16):
            ov0[e, pl.ds(j * 16, 16)] = zv
            ov1[e, pl.ds(j * 16, 16)] = zv
        return carry

    lax.fori_loop(0, KD, zov, 0)
    lane0 = lax.iota(jnp.int32, 16) == 0

    def lin_issue(ch, s):
        base = ebase + ch * KD
        pltpu.async_copy(dst_hbm.at[pl.ds(base, KD)], dstv[s], semL[s])
        pltpu.async_copy(w_hbm.at[pl.ds(base, KD)], wv[s], semL[s])

    def lin_wait(s):
        pltpu.make_async_copy(dst_hbm.at[pl.ds(0, KD)], dstv[s], semL[s]).wait()
        pltpu.make_async_copy(w_hbm.at[pl.ds(0, KD)], wv[s], semL[s]).wait()

    def compute(s):
        for base in (0, 16, KD - 16):
            wvec = wv[s][pl.ds(base, 16)]
            for l in range(16):
                smw[base + l] = wvec[l]
        dsts[s][pl.ds(0, 16)] = dstv[s][pl.ds(0, 16)]
        dsts[s][pl.ds(16, 16)] = dstv[s][pl.ds(16, 16)]
        dsts[s][pl.ds(KD - 16, 16)] = dstv[s][pl.ds(KD - 16, 16)]

        def edge(e, ecarry):
            ov[s][e, pl.ds(0, 16)] = jnp.where(lane0, smw[e], 0.0)
            return ecarry

        lax.fori_loop(0, KD, edge, 0)

    def scat_issue(s):
        pltpu.async_copy(ov[s], s_sp.at[dsts[s]], semS[s], add=True)

    def scat_wait(s):
        pltpu.make_async_copy(ov[s], s_sp.at[dsts[s]], semS[s]).wait()

    lin_issue(0, 0)
    lin_issue(1, 1)

    def piped(ch, s, wait_scat):
        lin_wait(s)
        if wait_scat:
            scat_wait(s)
        compute(s)
        scat_issue(s)
        lin_issue(ch + 2, s)

    piped(0, 0, False)
    piped(1, 1, False)

    def outer(g, carry):
        piped(2 + g * 2, 0, True)
        piped(3 + g * 2, 1, True)
        return carry

    lax.fori_loop(0, (NCHD - 4) // 2, outer, 0)
    lin_wait(0)
    scat_wait(0)
    compute(0)
    scat_issue(0)
    lin_wait(1)
    scat_wait(1)
    compute(1)
    scat_issue(1)
    scat_wait(0)
    scat_wait(1)

    plsc.subcore_barrier()

    for t in range(SPT // ZR):
        r0 = sid * SPT + t * ZR
        pltpu.sync_copy(s_sp.at[pl.ds(r0, ZR)], zb)
        pltpu.sync_copy(zb, out_hbm.at[cid, pl.ds(r0, ZR)])


BN = 1000
GN = N // BN
BE = 2000
GE = E // BE

_f32 = jnp.float32


def _dot(a, b):
    return jnp.dot(a, b, preferred_element_type=_f32)


def _h0_body(x_ref, w_ref, b_ref, o_ref):
    o_ref[...] = _dot(x_ref[...], w_ref[...]) + b_ref[...]


_tc_h0 = pl.pallas_call(
    _h0_body,
    grid=(GN,),
    in_specs=[pl.BlockSpec((BN, D), lambda i: (i, 0)),
              pl.BlockSpec((D, D), lambda i: (0, 0)),
              pl.BlockSpec((1, D), lambda i: (0, 0))],
    out_specs=pl.BlockSpec((BN, D), lambda i: (i, 0)),
    out_shape=jax.ShapeDtypeStruct((N, D), _f32),
)


def _ab_body(h_ref, sf_ref, ws_ref, wd_ref, wss_ref, wsd_ref, b1_ref,
             a_ref, b_ref):
    h = h_ref[...]
    sf = sf_ref[...]
    a_ref[...] = _dot(h, ws_ref[...]) + _dot(sf, wss_ref[...])
    b_ref[...] = _dot(h, wd_ref[...]) + _dot(sf, wsd_ref[...]) + b1_ref[...]


_tc_ab = pl.pallas_call(
    _ab_body,
    grid=(GN,),
    in_specs=[pl.BlockSpec((BN, D), lambda i: (i, 0)),
              pl.BlockSpec((BN, 8), lambda i: (i, 0)),
              pl.BlockSpec((D, D), lambda i: (0, 0)),
              pl.BlockSpec((D, D), lambda i: (0, 0)),
              pl.BlockSpec((8, D), lambda i: (0, 0)),
              pl.BlockSpec((8, D), lambda i: (0, 0)),
              pl.BlockSpec((1, D), lambda i: (0, 0))],
    out_specs=[pl.BlockSpec((BN, D), lambda i: (i, 0)),
               pl.BlockSpec((BN, D), lambda i: (i, 0))],
    out_shape=[jax.ShapeDtypeStruct((N, D), _f32),
               jax.ShapeDtypeStruct((N, D), _f32)],
)


def _eproj_body(ea_ref, w0_ref, w1_ref, w2_ref, c0_ref, c1_ref, c2_ref):
    ea = ea_ref[...]
    c0_ref[...] = _dot(ea, w0_ref[...])
    c1_ref[...] = _dot(ea, w1_ref[...])
    c2_ref[...] = _dot(ea, w2_ref[...])


_tc_eproj = pl.pallas_call(
    _eproj_body,
    grid=(GE,),
    in_specs=[pl.BlockSpec((BE, 16), lambda i: (i, 0)),
              pl.BlockSpec((16, D), lambda i: (0, 0)),
              pl.BlockSpec((16, D), lambda i: (0, 0)),
              pl.BlockSpec((16, D), lambda i: (0, 0))],
    out_specs=[pl.BlockSpec((BE, D), lambda i: (i, 0)),
               pl.BlockSpec((BE, D), lambda i: (i, 0)),
               pl.BlockSpec((BE, D), lambda i: (i, 0))],
    out_shape=[jax.ShapeDtypeStruct((E, D), _f32),
               jax.ShapeDtypeStruct((E, D), _f32),
               jax.ShapeDtypeStruct((E, D), _f32)],
)


def _upd_body(h_ref, sp_ref, dg_ref, w2_ref, b2_ref, u1h_ref, u1u_ref,
              ub1_ref, u2_ref, ub2_ref, o_ref):
    s = sp_ref[0] + sp_ref[1]
    deg = dg_ref[0, :, 0:1] + dg_ref[1, :, 0:1]
    upd = _dot(s, w2_ref[...]) + deg * b2_ref[...]
    t = _dot(h_ref[...], u1h_ref[...]) + _dot(upd, u1u_ref[...]) + ub1_ref[...]
    t = jnp.maximum(t, 0.0)
    o = _dot(t, u2_ref[...]) + ub2_ref[...]
    o_ref[...] = jnp.maximum(o, 0.0)


_tc_update = pl.pallas_call(
    _upd_body,
    grid=(GN,),
    in_specs=[pl.BlockSpec((BN, D), lambda i: (i, 0)),
              pl.BlockSpec((NC, BN, DS), lambda i: (0, i, 0)),
              pl.BlockSpec((NC, BN, DS), lambda i: (0, i, 0)),
              pl.BlockSpec((D, D), lambda i: (0, 0)),
              pl.BlockSpec((1, D), lambda i: (0, 0)),
              pl.BlockSpec((D, D), lambda i: (0, 0)),
              pl.BlockSpec((D, D), lambda i: (0, 0)),
              pl.BlockSpec((1, D), lambda i: (0, 0)),
              pl.BlockSpec((D, D), lambda i: (0, 0)),
              pl.BlockSpec((1, D), lambda i: (0, 0))],
    out_specs=pl.BlockSpec((BN, D), lambda i: (i, 0)),
    out_shape=jax.ShapeDtypeStruct((N, D), _f32),
)


def _rs_body(h_ref, o_ref):
    @pl.when(pl.program_id(0) == 0)
    def _():
        o_ref[...] = jnp.zeros_like(o_ref)

    o_ref[...] += jnp.sum(h_ref[...], axis=0, keepdims=True)


_tc_rowsum = pl.pallas_call(
    _rs_body,
    grid=(GN,),
    in_specs=[pl.BlockSpec((BN, D), lambda i: (i, 0))],
    out_specs=pl.BlockSpec((1, D), lambda i: (0, 0)),
    out_shape=jax.ShapeDtypeStruct((1, D), _f32),
)


def kernel(x, node_structural_feature, edge_attr, edge_weight, W0, b0,
           msgW1, msgb1, msgW2, msgb2, updW1, updb1, updW2, updb2,
           edge_index):
    src = edge_index[0]
    dst = edge_index[1]
    sfp = jnp.pad(node_structural_feature, ((0, 0), (0, 2)))

    h = _tc_h0(x, W0, b0.reshape(1, D))
    c_all = _tc_eproj(edge_attr,
                      msgW1[0, 268:284], msgW1[1, 268:284], msgW1[2, 268:284])
    degp = _sc_deg(dst, edge_weight)

    for i in range(LAYERS):
        ws = msgW1[i, 0:128]
        wd = msgW1[i, 128:256]
        wss = jnp.pad(msgW1[i, 256:262], ((0, 2), (0, 0)))
        wsd = jnp.pad(msgW1[i, 262:268], ((0, 2), (0, 0)))
        a, b = _tc_ab(h, sfp, ws, wd, wss, wsd, msgb1[i].reshape(1, D))
        sp = _sc_edge(a, b, c_all[i], src, dst, edge_weight)
        h = _tc_update(h, sp, degp, msgW2[i], msgb2[i].reshape(1, D),
                       updW1[i, 0:128], updW1[i, 128:256],
                       updb1[i].reshape(1, D), updW2[i],
                       updb2[i].reshape(1, D))

    graph_feature = _tc_rowsum(h)
    return graph_feature, h
